# baseline (speedup 1.0000x reference)
import functools
import jax
import jax.numpy as jnp
from jax import lax
from jax.experimental import pallas as pl
from jax.experimental.pallas import tpu as pltpu
from jax.experimental.pallas import tpu_sc as plsc

_NUM_ATOMS = 100000
_TOTAL = 32768
_B = 16
_NC = 2
_NS = 16
_L = 16
_NW = _NC * _NS
_TOK_W = _TOTAL // _NW
_GCH = 128
_NG = _TOK_W // _GCH
_CHW = 3136
_CHW_LAST = _NUM_ATOMS - _CHW
_PB = _B * _L

_NEG_INF = float("-inf")
_POS_INF = float("inf")


def _sc_body(phi_hbm, idx_hbm, cu_hbm, outmax_hbm, outmin_hbm,
             cu_v, idx_v, vals_v, minb_v, vec_v, pm_v, tseg_v,
             shmax, shmin, cmbmin_v, sem, sem_in, sem_min, sem_pub):
    cid = lax.axis_index("c")
    sid = lax.axis_index("s")
    wid = sid * _NC + cid
    base = wid * _TOK_W

    cp_idx = pltpu.async_copy(idx_hbm.at[pl.ds(base, _TOK_W)], idx_v, sem_in)
    moff = jnp.minimum(wid * _CHW, _CHW_LAST)
    cp_min = pltpu.async_copy(phi_hbm.at[pl.ds(moff, _CHW)], minb_v, sem_min)
    cp_cu = pltpu.async_copy(cu_hbm, cu_v.at[pl.ds(0, _B + 1)], sem_in)

    cp_idx.wait()
    cp_g = pltpu.async_copy(phi_hbm.at[idx_v], vals_v, sem)

    cp_min.wait()

    def nbody(j, acc):
        o = j * (4 * _L)
        acc = jnp.minimum(acc, minb_v[pl.ds(o, _L)])
        acc = jnp.minimum(acc, minb_v[pl.ds(o + _L, _L)])
        acc = jnp.minimum(acc, minb_v[pl.ds(o + 2 * _L, _L)])
        return jnp.minimum(acc, minb_v[pl.ds(o + 3 * _L, _L)])

    mn = lax.fori_loop(0, _CHW // (4 * _L), nbody,
                       jnp.full((_L,), _POS_INF, jnp.float32))
    vec_v[...] = mn
    pltpu.sync_copy(vec_v, shmin.at[pl.ds(sid * _L, _L)])

    cp_cu.wait()
    lane = lax.broadcasted_iota(jnp.int32, (_L,), 0)
    lov = cu_v[pl.ds(0, _L)]
    hiv = cu_v[pl.ds(1, _L)]
    cp_g.wait()

    pubs = []
    for b in range(_B):
        lo = lov[b]
        hi = hiv[b]
        s0 = jnp.maximum(lo, base)
        e0 = jnp.minimum(hi, base + _TOK_W)
        j0 = (s0 - base) // _L
        j1 = jnp.maximum(j0, (e0 - base + (_L - 1)) // _L)

        def mbody(j, acc, lo=lo, hi=hi):
            pos = base + j * _L + lane
            v = vals_v[pl.ds(j * _L, _L)]
            m = (pos >= lo) & (pos < hi)
            return jnp.maximum(acc, jnp.where(m, v, _NEG_INF))

        acc = lax.fori_loop(j0, j1, mbody, jnp.full((_L,), _NEG_INF, jnp.float32))
        pm_v[pl.ds(b * _L, _L)] = acc
        pubs.append(pltpu.async_copy(
            pm_v.at[pl.ds(b * _L, _L)],
            shmax.at[pl.ds(b * _NS * _L + sid * _L, _L)],
            sem_pub,
        ))
    for cp in pubs:
        cp.wait()

    plsc.subcore_barrier()

    pltpu.sync_copy(shmax.at[pl.ds(sid * _NS * _L, _NS * _L)], tseg_v)
    a = tseg_v[pl.ds(0, _L)]
    for r in range(1, _NS):
        a = jnp.maximum(a, tseg_v[pl.ds(r * _L, _L)])
    vec_v[...] = a
    pltpu.sync_copy(vec_v, outmax_hbm.at[pl.ds(cid * _PB + sid * _L, _L)])

    @pl.when(sid == 0)
    def _():
        pltpu.sync_copy(shmin, cmbmin_v)
        a = cmbmin_v[pl.ds(0, _L)]
        for r in range(1, _NS):
            a = jnp.minimum(a, cmbmin_v[pl.ds(r * _L, _L)])
        vec_v[...] = a
        pltpu.sync_copy(vec_v, outmin_hbm.at[pl.ds(cid * _L, _L)])


_sc_call = functools.partial(
    pl.kernel,
    out_type=[
        jax.ShapeDtypeStruct((_NC * _PB,), jnp.float32),
        jax.ShapeDtypeStruct((_NC * _L,), jnp.float32),
    ],
    scratch_types=[
        pltpu.VMEM((2 * _L,), jnp.int32),
        pltpu.VMEM((_TOK_W,), jnp.int32),
        pltpu.VMEM((_TOK_W,), jnp.float32),
        pltpu.VMEM((_CHW,), jnp.float32),
        pltpu.VMEM((_L,), jnp.float32),
        pltpu.VMEM((_PB,), jnp.float32),
        pltpu.VMEM((_NS * _L,), jnp.float32),
        pltpu.VMEM_SHARED((_B * _NS * _L,), jnp.float32),
        pltpu.VMEM_SHARED((_NS * _L,), jnp.float32),
        pltpu.VMEM((_NS * _L,), jnp.float32),
        pltpu.SemaphoreType.DMA,
        pltpu.SemaphoreType.DMA,
        pltpu.SemaphoreType.DMA,
        pltpu.SemaphoreType.DMA,
    ],
    mesh=plsc.VectorSubcoreMesh(core_axis_name="c", subcore_axis_name="s",
                                num_cores=_NC, num_subcores=_NS),
    name="aml_seg_max_sc",
)(_sc_body)


@jax.jit
def kernel(phi, flat_indices, cu_seqlens):
    phi = phi.astype(jnp.float32)
    idx = flat_indices.astype(jnp.int32)
    cu = cu_seqlens.astype(jnp.int32)
    outmax, outmin = _sc_call(phi, idx, cu)
    lengths = cu[1:] - cu[:-1]
    segmax = jnp.max(outmax.reshape(_NC, _B, _L), axis=(0, 2))
    traces = jnp.where(lengths == 0, jnp.min(outmin), segmax)
    return traces


# speedup vs baseline: 1.0148x; 1.0148x over previous
import functools
import jax
import jax.numpy as jnp
from jax import lax
from jax.experimental import pallas as pl
from jax.experimental.pallas import tpu as pltpu
from jax.experimental.pallas import tpu_sc as plsc

_NUM_ATOMS = 100000
_TOTAL = 32768
_B = 16
_NC = 2
_NS = 16
_L = 16
_NW = _NC * _NS
_TOK_W = _TOTAL // _NW
_GCH = 128
_NG = _TOK_W // _GCH
_CHW = 3136
_CHW_LAST = _NUM_ATOMS - _CHW
_PB = _B * _L
_CB = _PB + _L

_NEG_INF = float("-inf")
_POS_INF = float("inf")


def _sc_body(phi_hbm, idx_hbm, cu_hbm, out_hbm,
             cu_v, idx_v, vals_v, minb_v, vec_v, pm_v, tseg_v,
             shmax, shmin, cmbmin_v, sem, sem_in, sem_min, sem_pub):
    cid = lax.axis_index("c")
    sid = lax.axis_index("s")
    wid = sid * _NC + cid
    base = wid * _TOK_W

    cp_idx = pltpu.async_copy(idx_hbm.at[pl.ds(base, _TOK_W)], idx_v, sem_in)
    moff = jnp.minimum(wid * _CHW, _CHW_LAST)
    cp_min = pltpu.async_copy(phi_hbm.at[pl.ds(moff, _CHW)], minb_v, sem_min)
    cp_cu = pltpu.async_copy(cu_hbm, cu_v.at[pl.ds(0, _B + 1)], sem_in)

    cp_idx.wait()
    cp_g = pltpu.async_copy(phi_hbm.at[idx_v], vals_v, sem)

    cp_min.wait()

    def nbody(j, acc):
        o = j * (4 * _L)
        acc = jnp.minimum(acc, minb_v[pl.ds(o, _L)])
        acc = jnp.minimum(acc, minb_v[pl.ds(o + _L, _L)])
        acc = jnp.minimum(acc, minb_v[pl.ds(o + 2 * _L, _L)])
        return jnp.minimum(acc, minb_v[pl.ds(o + 3 * _L, _L)])

    mn = lax.fori_loop(0, _CHW // (4 * _L), nbody,
                       jnp.full((_L,), _POS_INF, jnp.float32))
    vec_v[...] = mn
    pltpu.sync_copy(vec_v, shmin.at[pl.ds(sid * _L, _L)])

    cp_cu.wait()
    lane = lax.broadcasted_iota(jnp.int32, (_L,), 0)
    lov = cu_v[pl.ds(0, _L)]
    hiv = cu_v[pl.ds(1, _L)]
    cp_g.wait()

    pubs = []
    for b in range(_B):
        lo = lov[b]
        hi = hiv[b]
        s0 = jnp.maximum(lo, base)
        e0 = jnp.minimum(hi, base + _TOK_W)
        j0 = (s0 - base) // _L
        j1 = jnp.maximum(j0, (e0 - base + (_L - 1)) // _L)

        def mbody(j, acc, lo=lo, hi=hi):
            pos = base + j * _L + lane
            v = vals_v[pl.ds(j * _L, _L)]
            m = (pos >= lo) & (pos < hi)
            return jnp.maximum(acc, jnp.where(m, v, _NEG_INF))

        acc = lax.fori_loop(j0, j1, mbody, jnp.full((_L,), _NEG_INF, jnp.float32))
        pm_v[pl.ds(b * _L, _L)] = acc
        pubs.append(pltpu.async_copy(
            pm_v.at[pl.ds(b * _L, _L)],
            shmax.at[pl.ds(b * _NS * _L + sid * _L, _L)],
            sem_pub,
        ))
    for cp in pubs:
        cp.wait()

    plsc.subcore_barrier()

    pltpu.sync_copy(shmax.at[pl.ds(sid * _NS * _L, _NS * _L)], tseg_v)
    a = tseg_v[pl.ds(0, _L)]
    for r in range(1, _NS):
        a = jnp.maximum(a, tseg_v[pl.ds(r * _L, _L)])
    vec_v[...] = a
    pltpu.sync_copy(vec_v, out_hbm.at[pl.ds(cid * _CB + sid * _L, _L)])

    @pl.when(sid == 0)
    def _():
        pltpu.sync_copy(shmin, cmbmin_v)
        a = cmbmin_v[pl.ds(0, _L)]
        for r in range(1, _NS):
            a = jnp.minimum(a, cmbmin_v[pl.ds(r * _L, _L)])
        vec_v[...] = a
        pltpu.sync_copy(vec_v, out_hbm.at[pl.ds(cid * _CB + _PB, _L)])


_sc_call = functools.partial(
    pl.kernel,
    out_type=jax.ShapeDtypeStruct((_NC * _CB,), jnp.float32),
    scratch_types=[
        pltpu.VMEM((2 * _L,), jnp.int32),
        pltpu.VMEM((_TOK_W,), jnp.int32),
        pltpu.VMEM((_TOK_W,), jnp.float32),
        pltpu.VMEM((_CHW,), jnp.float32),
        pltpu.VMEM((_L,), jnp.float32),
        pltpu.VMEM((_PB,), jnp.float32),
        pltpu.VMEM((_NS * _L,), jnp.float32),
        pltpu.VMEM_SHARED((_B * _NS * _L,), jnp.float32),
        pltpu.VMEM_SHARED((_NS * _L,), jnp.float32),
        pltpu.VMEM((_NS * _L,), jnp.float32),
        pltpu.SemaphoreType.DMA,
        pltpu.SemaphoreType.DMA,
        pltpu.SemaphoreType.DMA,
        pltpu.SemaphoreType.DMA,
    ],
    mesh=plsc.VectorSubcoreMesh(core_axis_name="c", subcore_axis_name="s",
                                num_cores=_NC, num_subcores=_NS),
    name="aml_seg_max_sc",
)(_sc_body)


@jax.jit
def kernel(phi, flat_indices, cu_seqlens):
    phi = phi.astype(jnp.float32)
    idx = flat_indices.astype(jnp.int32)
    cu = cu_seqlens.astype(jnp.int32)
    out = _sc_call(phi, idx, cu)
    out = out.reshape(_NC, _CB)
    lengths = cu[1:] - cu[:-1]
    segmax = jnp.max(out[:, :_PB].reshape(_NC, _B, _L), axis=(0, 2))
    gmin = jnp.min(out[:, _PB:])
    traces = jnp.where(lengths == 0, gmin, segmax)
    return traces


# speedup vs baseline: 1.0153x; 1.0005x over previous
import functools
import jax
import jax.numpy as jnp
from jax import lax
from jax.experimental import pallas as pl
from jax.experimental.pallas import tpu as pltpu
from jax.experimental.pallas import tpu_sc as plsc

_NUM_ATOMS = 100000
_TOTAL = 32768
_B = 16
_NC = 2
_NS = 16
_L = 16
_NW = _NC * _NS
_TOK_W = _TOTAL // _NW
_GCH = 128
_NG = _TOK_W // _GCH
_CHW = 3136
_CHW_LAST = _NUM_ATOMS - _CHW
_PB = _B * _L
_CB = _PB + _L

_NEG_INF = float("-inf")
_POS_INF = float("inf")


def _sc_body(phi_hbm, idx_hbm, cu_hbm, out_hbm,
             cu_v, idx_v, vals_v, minb_v, vec_v, pm_v, tseg_v,
             shmax, shmin, cmbmin_v, sem, sem_in, sem_min, sem_pub):
    cid = lax.axis_index("c")
    sid = lax.axis_index("s")
    wid = sid * _NC + cid
    base = wid * _TOK_W

    _H = _TOK_W // 2
    cp_i0 = pltpu.async_copy(idx_hbm.at[pl.ds(base, _H)],
                             idx_v.at[pl.ds(0, _H)], sem_in)
    cp_i1 = pltpu.async_copy(idx_hbm.at[pl.ds(base + _H, _H)],
                             idx_v.at[pl.ds(_H, _H)], sem_in)
    moff = jnp.minimum(wid * _CHW, _CHW_LAST)
    cp_min = pltpu.async_copy(phi_hbm.at[pl.ds(moff, _CHW)], minb_v, sem_min)
    cp_cu = pltpu.async_copy(cu_hbm, cu_v.at[pl.ds(0, _B + 1)], sem_in)

    cp_i0.wait()
    cp_g0 = pltpu.async_copy(phi_hbm.at[idx_v.at[pl.ds(0, _H)]],
                             vals_v.at[pl.ds(0, _H)], sem)
    cp_i1.wait()
    cp_g1 = pltpu.async_copy(phi_hbm.at[idx_v.at[pl.ds(_H, _H)]],
                             vals_v.at[pl.ds(_H, _H)], sem)

    cp_min.wait()

    def nbody(j, acc):
        o = j * (4 * _L)
        acc = jnp.minimum(acc, minb_v[pl.ds(o, _L)])
        acc = jnp.minimum(acc, minb_v[pl.ds(o + _L, _L)])
        acc = jnp.minimum(acc, minb_v[pl.ds(o + 2 * _L, _L)])
        return jnp.minimum(acc, minb_v[pl.ds(o + 3 * _L, _L)])

    mn = lax.fori_loop(0, _CHW // (4 * _L), nbody,
                       jnp.full((_L,), _POS_INF, jnp.float32))
    vec_v[...] = mn
    pltpu.sync_copy(vec_v, shmin.at[pl.ds(sid * _L, _L)])

    cp_cu.wait()
    lane = lax.broadcasted_iota(jnp.int32, (_L,), 0)
    lov = cu_v[pl.ds(0, _L)]
    hiv = cu_v[pl.ds(1, _L)]
    cp_g0.wait()
    cp_g1.wait()

    pubs = []
    for b in range(_B):
        lo = lov[b]
        hi = hiv[b]
        s0 = jnp.maximum(lo, base)
        e0 = jnp.minimum(hi, base + _TOK_W)
        j0 = (s0 - base) // _L
        j1 = jnp.maximum(j0, (e0 - base + (_L - 1)) // _L)

        def mbody(j, acc, lo=lo, hi=hi):
            pos = base + j * _L + lane
            v = vals_v[pl.ds(j * _L, _L)]
            m = (pos >= lo) & (pos < hi)
            return jnp.maximum(acc, jnp.where(m, v, _NEG_INF))

        acc = lax.fori_loop(j0, j1, mbody, jnp.full((_L,), _NEG_INF, jnp.float32))
        pm_v[pl.ds(b * _L, _L)] = acc
        pubs.append(pltpu.async_copy(
            pm_v.at[pl.ds(b * _L, _L)],
            shmax.at[pl.ds(b * _NS * _L + sid * _L, _L)],
            sem_pub,
        ))
    for cp in pubs:
        cp.wait()

    plsc.subcore_barrier()

    pltpu.sync_copy(shmax.at[pl.ds(sid * _NS * _L, _NS * _L)], tseg_v)
    a = tseg_v[pl.ds(0, _L)]
    for r in range(1, _NS):
        a = jnp.maximum(a, tseg_v[pl.ds(r * _L, _L)])
    vec_v[...] = a
    pltpu.sync_copy(vec_v, out_hbm.at[pl.ds(cid * _CB + sid * _L, _L)])

    @pl.when(sid == 0)
    def _():
        pltpu.sync_copy(shmin, cmbmin_v)
        a = cmbmin_v[pl.ds(0, _L)]
        for r in range(1, _NS):
            a = jnp.minimum(a, cmbmin_v[pl.ds(r * _L, _L)])
        vec_v[...] = a
        pltpu.sync_copy(vec_v, out_hbm.at[pl.ds(cid * _CB + _PB, _L)])


_sc_call = functools.partial(
    pl.kernel,
    out_type=jax.ShapeDtypeStruct((_NC * _CB,), jnp.float32),
    scratch_types=[
        pltpu.VMEM((2 * _L,), jnp.int32),
        pltpu.VMEM((_TOK_W,), jnp.int32),
        pltpu.VMEM((_TOK_W,), jnp.float32),
        pltpu.VMEM((_CHW,), jnp.float32),
        pltpu.VMEM((_L,), jnp.float32),
        pltpu.VMEM((_PB,), jnp.float32),
        pltpu.VMEM((_NS * _L,), jnp.float32),
        pltpu.VMEM_SHARED((_B * _NS * _L,), jnp.float32),
        pltpu.VMEM_SHARED((_NS * _L,), jnp.float32),
        pltpu.VMEM((_NS * _L,), jnp.float32),
        pltpu.SemaphoreType.DMA,
        pltpu.SemaphoreType.DMA,
        pltpu.SemaphoreType.DMA,
        pltpu.SemaphoreType.DMA,
    ],
    mesh=plsc.VectorSubcoreMesh(core_axis_name="c", subcore_axis_name="s",
                                num_cores=_NC, num_subcores=_NS),
    name="aml_seg_max_sc",
)(_sc_body)


@jax.jit
def kernel(phi, flat_indices, cu_seqlens):
    phi = phi.astype(jnp.float32)
    idx = flat_indices.astype(jnp.int32)
    cu = cu_seqlens.astype(jnp.int32)
    out = _sc_call(phi, idx, cu)
    out = out.reshape(_NC, _CB)
    lengths = cu[1:] - cu[:-1]
    segmax = jnp.max(out[:, :_PB].reshape(_NC, _B, _L), axis=(0, 2))
    gmin = jnp.min(out[:, _PB:])
    traces = jnp.where(lengths == 0, gmin, segmax)
    return traces


# speedup vs baseline: 1.1097x; 1.0930x over previous
import functools
import jax
import jax.numpy as jnp
from jax import lax
from jax.experimental import pallas as pl
from jax.experimental.pallas import tpu as pltpu
from jax.experimental.pallas import tpu_sc as plsc

_NUM_ATOMS = 100000
_TOTAL = 32768
_B = 16
_NC = 2
_NS = 16
_L = 16
_BC = _B // _NC
_TOK_W = 2048
_H = _TOK_W // 2
_CHW = 6272
_CHW_LAST = _NUM_ATOMS - _CHW

_NEG_INF = float("-inf")
_POS_INF = float("inf")


def _sc_body(phi_hbm, idx_hbm, cu_hbm, out_hbm,
             cu_v, idx_v, vals_v, minb_v, vec_v, pm_v, tseg_v, bufa_v, bufb_v,
             shmax, shmin, shfin, sem, sem_in, sem_min, sem_pub):
    cid = lax.axis_index("c")
    sid = lax.axis_index("s")
    lane = lax.broadcasted_iota(jnp.int32, (_L,), 0)
    ninf = jnp.full((_L,), _NEG_INF, jnp.float32)
    pinf = jnp.full((_L,), _POS_INF, jnp.float32)

    moff = jnp.minimum(sid * _CHW, _CHW_LAST)
    cp_min = pltpu.async_copy(phi_hbm.at[pl.ds(moff, _CHW)], minb_v, sem_min)
    cp_cu = pltpu.async_copy(cu_hbm, cu_v.at[pl.ds(0, _B + 1)], sem_in)
    cp_cu.wait()

    cuv0 = cu_v[pl.ds(0, _L)]
    t0 = jnp.where(cid == 0, cuv0[0], cuv0[_BC])
    t1 = jnp.where(cid == 0, cuv0[_BC], jnp.int32(_TOTAL))
    t0a = (t0 // _L) * _L
    stride = ((t1 - t0a + (_NS * _L - 1)) // (_NS * _L)) * _L
    off = jnp.minimum(t0a + sid * stride, _TOTAL - _TOK_W)

    cp_i0 = pltpu.async_copy(idx_hbm.at[pl.ds(off, _H)],
                             idx_v.at[pl.ds(0, _H)], sem_in)
    cp_i1 = pltpu.async_copy(idx_hbm.at[pl.ds(off + _H, _H)],
                             idx_v.at[pl.ds(_H, _H)], sem_in)
    cp_i0.wait()
    cp_g0 = pltpu.async_copy(phi_hbm.at[idx_v.at[pl.ds(0, _H)]],
                             vals_v.at[pl.ds(0, _H)], sem)
    cp_i1.wait()
    cp_g1 = pltpu.async_copy(phi_hbm.at[idx_v.at[pl.ds(_H, _H)]],
                             vals_v.at[pl.ds(_H, _H)], sem)

    cp_min.wait()

    def nbody(j, acc):
        o = j * (4 * _L)
        acc = jnp.minimum(acc, minb_v[pl.ds(o, _L)])
        acc = jnp.minimum(acc, minb_v[pl.ds(o + _L, _L)])
        acc = jnp.minimum(acc, minb_v[pl.ds(o + 2 * _L, _L)])
        return jnp.minimum(acc, minb_v[pl.ds(o + 3 * _L, _L)])

    mn = lax.fori_loop(0, _CHW // (4 * _L), nbody, pinf)
    vec_v[...] = mn
    pltpu.sync_copy(vec_v, shmin.at[pl.ds(sid * _L, _L)])

    cp_g0.wait()
    cp_g1.wait()

    pubs = []
    for b in range(_BC):
        cub = cu_v[pl.ds(_BC * cid + b, _L)]
        lo = cub[0]
        hi = cub[1]
        s0 = jnp.maximum(lo, off)
        e0 = jnp.minimum(hi, off + _TOK_W)
        j0 = (s0 - off) // _L
        j1 = jnp.maximum(j0, (e0 - off + (_L - 1)) // _L)

        def mbody(j, acc, lo=lo, hi=hi):
            pos = off + j * _L + lane
            v = vals_v[pl.ds(j * _L, _L)]
            m = (pos >= lo) & (pos < hi)
            return jnp.maximum(acc, jnp.where(m, v, _NEG_INF))

        acc = lax.fori_loop(j0, j1, mbody, ninf)
        pm_v[pl.ds(b * _L, _L)] = acc
        pubs.append(pltpu.async_copy(
            pm_v.at[pl.ds(b * _L, _L)],
            shmax.at[pl.ds(b * _NS * _L + sid * _L, _L)],
            sem_pub,
        ))
    for cp in pubs:
        cp.wait()

    plsc.subcore_barrier()

    @pl.when(sid < _BC)
    def _():
        pltpu.sync_copy(shmax.at[pl.ds(sid * _NS * _L, _NS * _L)], tseg_v)
        a = tseg_v[pl.ds(0, _L)]
        for r in range(1, _NS):
            a = jnp.maximum(a, tseg_v[pl.ds(r * _L, _L)])
        bufa_v[pl.ds(_L, _L)] = ninf
        for k in (8, 4, 2, 1):
            bufa_v[pl.ds(0, _L)] = a
            a = jnp.maximum(a, bufa_v[pl.ds(k, _L)])
        rot = jnp.where(lane == 0, a, _NEG_INF)
        bufb_v[pl.ds(0, _L)] = ninf
        bufb_v[pl.ds(2 * _L, _L)] = ninf
        bufb_v[pl.ds(_L, _L)] = rot
        vec_v[...] = bufb_v[pl.ds(_L - sid, _L)]
        pltpu.sync_copy(vec_v, shfin.at[pl.ds(sid * _L, _L)])

    plsc.subcore_barrier()

    @pl.when(sid == 0)
    def _():
        pltpu.sync_copy(shfin, pm_v)
        segv = pm_v[pl.ds(0, _L)]
        for r in range(1, _BC):
            segv = jnp.maximum(segv, pm_v[pl.ds(r * _L, _L)])

        pltpu.sync_copy(shmin, tseg_v)
        g = tseg_v[pl.ds(0, _L)]
        for r in range(1, _NS):
            g = jnp.minimum(g, tseg_v[pl.ds(r * _L, _L)])
        bufa_v[pl.ds(_L, _L)] = pinf
        for k in (8, 4, 2, 1):
            bufa_v[pl.ds(0, _L)] = g
            g = jnp.minimum(g, bufa_v[pl.ds(k, _L)])
        g = jnp.where(lane == 0, g, _POS_INF)
        bufb_v[pl.ds(0, _L)] = pinf
        for k in (1, 2, 4, 8):
            bufb_v[pl.ds(_L, _L)] = g
            g = jnp.minimum(g, bufb_v[pl.ds(_L - k, _L)])

        lov = cu_v[pl.ds(_BC * cid, _L)]
        hiv = cu_v[pl.ds(_BC * cid + 1, _L)]
        vec_v[...] = jnp.where(hiv - lov == 0, g, segv)
        pltpu.sync_copy(vec_v.at[pl.ds(0, _BC)],
                        out_hbm.at[pl.ds(_BC * cid, _BC)])


_sc_call = functools.partial(
    pl.kernel,
    out_type=jax.ShapeDtypeStruct((_B,), jnp.float32),
    scratch_types=[
        pltpu.VMEM((2 * _L,), jnp.int32),
        pltpu.VMEM((_TOK_W,), jnp.int32),
        pltpu.VMEM((_TOK_W,), jnp.float32),
        pltpu.VMEM((_CHW,), jnp.float32),
        pltpu.VMEM((_L,), jnp.float32),
        pltpu.VMEM((_BC * _L,), jnp.float32),
        pltpu.VMEM((_NS * _L,), jnp.float32),
        pltpu.VMEM((2 * _L,), jnp.float32),
        pltpu.VMEM((3 * _L,), jnp.float32),
        pltpu.VMEM_SHARED((_BC * _NS * _L,), jnp.float32),
        pltpu.VMEM_SHARED((_NS * _L,), jnp.float32),
        pltpu.VMEM_SHARED((_BC * _L,), jnp.float32),
        pltpu.SemaphoreType.DMA,
        pltpu.SemaphoreType.DMA,
        pltpu.SemaphoreType.DMA,
        pltpu.SemaphoreType.DMA,
    ],
    mesh=plsc.VectorSubcoreMesh(core_axis_name="c", subcore_axis_name="s",
                                num_cores=_NC, num_subcores=_NS),
    name="aml_seg_max_sc",
)(_sc_body)


@jax.jit
def kernel(phi, flat_indices, cu_seqlens):
    phi = phi.astype(jnp.float32)
    idx = flat_indices.astype(jnp.int32)
    cu = cu_seqlens.astype(jnp.int32)
    return _sc_call(phi, idx, cu)
